# baseline (speedup 1.0000x reference)
import functools

import jax
import jax.numpy as jnp
from jax import lax
from jax.experimental import pallas as pl
from jax.experimental.pallas import tpu as pltpu
from jax.experimental.pallas import tpu_sc as plsc


def kernel(x, y, name_embedding, value_table, mask_embedding):
    B, L = x.shape
    V, D = value_table.shape
    LP = 104
    DP = 2 * D
    NW = 32
    BPW = B // NW
    CB = 8
    NCH = BPW // CB
    ND = D // 16

    mesh = plsc.VectorSubcoreMesh(core_axis_name="c", subcore_axis_name="s")

    @functools.partial(
        pl.kernel,
        mesh=mesh,
        compiler_params=pltpu.CompilerParams(use_tc_tiling_on_sc=False,
                                             needs_layout_passes=False),
        out_type=jax.ShapeDtypeStruct((B, LP, DP), jnp.float32),
        scratch_types=[
            pltpu.VMEM((2, CB, L), jnp.int32),
            pltpu.VMEM((2, CB, L, D), jnp.float32),
            pltpu.VMEM((L, D), jnp.float32),
            pltpu.VMEM((D,), jnp.float32),
            pltpu.VMEM((BPW + 16,), jnp.int32),
            pltpu.SemaphoreType.DMA,
            pltpu.SemaphoreType.DMA,
        ],
    )
    def run(x_hbm, y_hbm, name_hbm, table_hbm, mask_hbm, out_hbm,
            idx2, rows2, name_v, mask_v, y_vmem, gsem, wsem):
        wid = lax.axis_index("s") * 2 + lax.axis_index("c")
        bbase = wid * BPW
        pltpu.sync_copy(name_hbm, name_v)
        pltpu.sync_copy(mask_hbm, mask_v)
        pltpu.sync_copy(y_hbm.at[pl.ds(bbase, BPW)], y_vmem.at[pl.ds(0, BPW)])
        lane = lax.iota(jnp.int32, 16)

        dummy_rows_hbm = out_hbm.at[pl.ds(0, CB), pl.ds(0, L), pl.ds(0, D)]

        def stage_and_gather(c, par):
            b0 = bbase + c * CB
            pltpu.sync_copy(x_hbm.at[pl.ds(b0, CB)], idx2.at[par])
            for j in range(CB):
                pltpu.async_copy(table_hbm.at[idx2.at[par, j]],
                                 rows2.at[par, j], gsem)

        stage_and_gather(0, 0)

        def chunk_body(c, carry):
            par = c & 1
            b0 = bbase + c * CB
            pltpu.make_async_copy(dummy_rows_hbm, rows2.at[par], gsem).wait()

            @pl.when(c > 0)
            def _():
                pltpu.make_async_copy(
                    rows2.at[1 - par], dummy_rows_hbm, wsem).wait()

            @pl.when(c < NCH - 1)
            def _():
                stage_and_gather(c + 1, 1 - par)

            def add_l(l, carry2):
                nm = [name_v[l, pl.ds(16 * d, 16)] for d in range(ND)]
                for b in range(CB):
                    for d in range(ND):
                        rows2[par, b, l, pl.ds(16 * d, 16)] = (
                            rows2[par, b, l, pl.ds(16 * d, 16)] + nm[d])
                return carry2

            lax.fori_loop(0, L, add_l, 0)

            y16 = y_vmem[pl.ds(c * CB, 16)]
            for b in range(CB):
                yb = jnp.max(jnp.where(lane == b, y16, 0))
                for d in range(ND):
                    rows2[par, b, yb, pl.ds(16 * d, 16)] = (
                        mask_v[pl.ds(16 * d, 16)]
                        + name_v[yb, pl.ds(16 * d, 16)])

            pltpu.async_copy(
                rows2.at[par],
                out_hbm.at[pl.ds(b0, CB), pl.ds(0, L), pl.ds(0, D)], wsem)
            return carry

        lax.fori_loop(0, NCH, chunk_body, 0)
        pltpu.make_async_copy(
            rows2.at[(NCH - 1) & 1], dummy_rows_hbm, wsem).wait()

    out = run(x, y, name_embedding, value_table, mask_embedding)
    return out[:, :L, :D]


# speedup vs baseline: 1.6322x; 1.6322x over previous
import functools

import jax
import jax.numpy as jnp
from jax import lax
from jax.experimental import pallas as pl
from jax.experimental.pallas import tpu as pltpu
from jax.experimental.pallas import tpu_sc as plsc


def kernel(x, y, name_embedding, value_table, mask_embedding):
    B, L = x.shape
    V, D = value_table.shape
    LP = 104
    DP = 2 * D
    NW = 32
    BPW = B // NW
    CB = 8
    NCH = BPW // CB
    NP = NCH // 2
    R = CB * L
    ND = D // 16

    mesh = plsc.VectorSubcoreMesh(core_axis_name="c", subcore_axis_name="s")

    @functools.partial(
        pl.kernel,
        mesh=mesh,
        compiler_params=pltpu.CompilerParams(use_tc_tiling_on_sc=False,
                                             needs_layout_passes=False),
        out_type=jax.ShapeDtypeStruct((B, LP, DP), jnp.float32),
        scratch_types=[
            pltpu.VMEM((CB, L), jnp.int32),
            pltpu.VMEM((CB, L), jnp.int32),
            pltpu.VMEM((R, D), jnp.float32),
            pltpu.VMEM((R, D), jnp.float32),
            pltpu.VMEM((L, D), jnp.float32),
            pltpu.VMEM((D,), jnp.float32),
            pltpu.VMEM((BPW + 16,), jnp.int32),
            pltpu.SemaphoreType.DMA,
        ],
    )
    def run(x_hbm, y_hbm, name_hbm, table_hbm, mask_hbm, out_hbm,
            idx_a, idx_b, rows_a, rows_b, name_v, mask_v, y_vmem, gsem):
        wid = lax.axis_index("s") * 2 + lax.axis_index("c")
        bbase = wid * BPW
        pltpu.sync_copy(name_hbm, name_v)
        pltpu.sync_copy(mask_hbm, mask_v)
        pltpu.sync_copy(y_hbm.at[pl.ds(bbase, BPW)], y_vmem.at[pl.ds(0, BPW)])
        lane = lax.iota(jnp.int32, 16)
        dummy_rows_hbm = out_hbm.at[0, pl.ds(0, L), pl.ds(0, D)]

        def stage_and_gather(c, idx_v, rows_v):
            b0 = bbase + c * CB
            pltpu.sync_copy(x_hbm.at[pl.ds(b0, CB)], idx_v)
            for j in range(CB):
                pltpu.async_copy(table_hbm.at[idx_v.at[j]],
                                 rows_v.at[pl.ds(j * L, L)], gsem)

        def process(c, idx_v, rows_v):
            b0 = bbase + c * CB
            for j in range(CB):
                pltpu.make_async_copy(
                    dummy_rows_hbm, rows_v.at[pl.ds(j * L, L)], gsem).wait()

            def add_l(l, carry2):
                nm = [name_v[l, pl.ds(16 * d, 16)] for d in range(ND)]
                for b in range(CB):
                    r = b * L + l
                    for d in range(ND):
                        rows_v[r, pl.ds(16 * d, 16)] = (
                            rows_v[r, pl.ds(16 * d, 16)] + nm[d])
                return carry2

            lax.fori_loop(0, L, add_l, 0)

            y16 = y_vmem[pl.ds(c * CB, 16)]
            for b in range(CB):
                yb = jnp.max(jnp.where(lane == b, y16, 0))
                r = b * L + yb
                for d in range(ND):
                    rows_v[r, pl.ds(16 * d, 16)] = (
                        mask_v[pl.ds(16 * d, 16)]
                        + name_v[yb, pl.ds(16 * d, 16)])

            for j in range(CB):
                pltpu.sync_copy(
                    rows_v.at[pl.ds(j * L, L)],
                    out_hbm.at[b0 + j, pl.ds(0, L), pl.ds(0, D)])

        stage_and_gather(0, idx_a, rows_a)

        def pair_body(c3, carry):
            c = 2 * c3
            stage_and_gather(c + 1, idx_b, rows_b)
            process(c, idx_a, rows_a)

            @pl.when(c3 < NP - 1)
            def _():
                stage_and_gather(c + 2, idx_a, rows_a)

            process(c + 1, idx_b, rows_b)
            return carry

        lax.fori_loop(0, NP, pair_body, 0)

    out = run(x, y, name_embedding, value_table, mask_embedding)
    return out[:, :L, :D]


# speedup vs baseline: 1.6914x; 1.0363x over previous
import functools

import jax
import jax.numpy as jnp
from jax import lax
from jax.experimental import pallas as pl
from jax.experimental.pallas import tpu as pltpu
from jax.experimental.pallas import tpu_sc as plsc


def kernel(x, y, name_embedding, value_table, mask_embedding):
    B, L = x.shape
    V, D = value_table.shape
    LP = 104
    DP = 2 * D
    NW = 32
    BPW = B // NW
    CB = 4
    NCH = BPW // CB
    NQ = NCH // 4
    R = CB * L
    ND = D // 16

    mesh = plsc.VectorSubcoreMesh(core_axis_name="c", subcore_axis_name="s")

    @functools.partial(
        pl.kernel,
        mesh=mesh,
        compiler_params=pltpu.CompilerParams(use_tc_tiling_on_sc=False,
                                             needs_layout_passes=False),
        out_type=jax.ShapeDtypeStruct((B, LP, DP), jnp.float32),
        scratch_types=[
            pltpu.VMEM((CB, L), jnp.int32),
            pltpu.VMEM((CB, L), jnp.int32),
            pltpu.VMEM((CB, L), jnp.int32),
            pltpu.VMEM((CB, L), jnp.int32),
            pltpu.VMEM((CB, L, D), jnp.float32),
            pltpu.VMEM((CB, L, D), jnp.float32),
            pltpu.VMEM((CB, L, D), jnp.float32),
            pltpu.VMEM((CB, L, D), jnp.float32),
            pltpu.VMEM((L, D), jnp.float32),
            pltpu.VMEM((D,), jnp.float32),
            pltpu.VMEM((BPW + 16,), jnp.int32),
            pltpu.SemaphoreType.DMA,
            pltpu.SemaphoreType.DMA,
            pltpu.SemaphoreType.DMA,
            pltpu.SemaphoreType.DMA,
            pltpu.SemaphoreType.DMA,
            pltpu.SemaphoreType.DMA,
            pltpu.SemaphoreType.DMA,
            pltpu.SemaphoreType.DMA,
        ],
    )
    def run(x_hbm, y_hbm, name_hbm, table_hbm, mask_hbm, out_hbm,
            idx0, idx1, idx2, idx3, rows0, rows1, rows2, rows3,
            name_v, mask_v, y_vmem,
            gsem0, gsem1, gsem2, gsem3, wsem0, wsem1, wsem2, wsem3):
        wid = lax.axis_index("s") * 2 + lax.axis_index("c")
        bbase = wid * BPW
        idx_bufs = [idx0, idx1, idx2, idx3]
        row_bufs = [rows0, rows1, rows2, rows3]
        gsems = [gsem0, gsem1, gsem2, gsem3]
        wsems = [wsem0, wsem1, wsem2, wsem3]
        pltpu.sync_copy(name_hbm, name_v)
        pltpu.sync_copy(mask_hbm, mask_v)
        pltpu.sync_copy(y_hbm.at[pl.ds(bbase, BPW)], y_vmem.at[pl.ds(0, BPW)])
        lane = lax.iota(jnp.int32, 16)
        dummy_chunk_hbm = out_hbm.at[pl.ds(0, CB), pl.ds(0, L), pl.ds(0, D)]

        def stage_and_gather(c, q):
            b0 = bbase + c * CB
            pltpu.sync_copy(x_hbm.at[pl.ds(b0, CB)], idx_bufs[q])
            for j in range(CB):
                pltpu.async_copy(table_hbm.at[idx_bufs[q].at[j]],
                                 row_bufs[q].at[j], gsems[q])

        def drain_write(q):
            pltpu.make_async_copy(row_bufs[q], dummy_chunk_hbm, wsems[q]).wait()

        def process(c, q):
            rows_v = row_bufs[q]
            b0 = bbase + c * CB
            pltpu.make_async_copy(dummy_chunk_hbm, rows_v, gsems[q]).wait()

            def add_l(l, carry2):
                nm = [name_v[l, pl.ds(16 * d, 16)] for d in range(ND)]
                for b in range(CB):
                    for d in range(ND):
                        rows_v[b, l, pl.ds(16 * d, 16)] = (
                            rows_v[b, l, pl.ds(16 * d, 16)] + nm[d])
                return carry2

            lax.fori_loop(0, L, add_l, 0)

            y16 = y_vmem[pl.ds(c * CB, 16)]
            for b in range(CB):
                yb = jnp.max(jnp.where(lane == b, y16, 0))
                for d in range(ND):
                    rows_v[b, yb, pl.ds(16 * d, 16)] = (
                        mask_v[pl.ds(16 * d, 16)]
                        + name_v[yb, pl.ds(16 * d, 16)])

            for j in range(CB):
                pltpu.async_copy(
                    rows_v.at[j],
                    out_hbm.at[b0 + j, pl.ds(0, L), pl.ds(0, D)], wsems[q])

        stage_and_gather(0, 0)
        stage_and_gather(1, 1)

        def ring_body(k, carry):
            c = 4 * k
            for h in range(4):
                process(c + h, h)
                if h >= 2:
                    drain_write((h + 2) % 4)
                else:
                    @pl.when(k > 0)
                    def _():
                        drain_write((h + 2) % 4)

                @pl.when(c + h + 2 < NCH)
                def _():
                    stage_and_gather(c + h + 2, (h + 2) % 4)
            return carry

        lax.fori_loop(0, NQ, ring_body, 0)
        drain_write(2)
        drain_write(3)

    out = run(x, y, name_embedding, value_table, mask_embedding)
    return out[:, :L, :D]
